# Initial kernel scaffold; baseline (speedup 1.0000x reference)
import jax
import jax.numpy as jnp
from jax.experimental import pallas as pl


def kernel(pred, target):
    raise NotImplementedError("write your pallas kernel here")


# speedup vs baseline: 2.0485x; 2.0485x over previous
import functools
import math

import jax
import jax.numpy as jnp
from jax.experimental import pallas as pl
from jax.experimental.pallas import tpu as pltpu

VOCAB = 100000
SMOOTHING = 0.1
PADDING_IDX = 0
CONFIDENCE = 1.0 - SMOOTHING
SV = SMOOTHING / (VOCAB - 2)
C_ENT = SMOOTHING * math.log(SV) + CONFIDENCE * math.log(CONFIDENCE)

BV = 2048


def _loss_kernel(tgt_ref, pred_ref, out_ref, m_ref, s_ref, sum_ref, pt_ref,
                 p0_ref, *, nb, vocab):
    i = pl.program_id(0)
    x = pred_ref[...]
    n = x.shape[0]

    cols = jax.lax.broadcasted_iota(jnp.int32, (1, BV), 1) + i * BV
    valid = cols < vocab

    @pl.when(i == 0)
    def _init():
        m_ref[...] = jnp.full((n, 1), -jnp.inf, jnp.float32)
        s_ref[...] = jnp.zeros((n, 1), jnp.float32)
        sum_ref[...] = jnp.zeros((n, 1), jnp.float32)
        pt_ref[...] = jnp.zeros((n, 1), jnp.float32)
        p0_ref[...] = x[:, 0:1]

    xm = jnp.where(valid, x, -jnp.inf)
    bmax = jnp.max(xm, axis=1, keepdims=True)
    m_new = jnp.maximum(m_ref[...], bmax)
    alpha = jnp.exp(m_ref[...] - m_new)
    bexp = jnp.sum(jnp.exp(xm - m_new), axis=1, keepdims=True)
    s_new = s_ref[...] * alpha + bexp
    m_ref[...] = m_new
    s_ref[...] = s_new

    sum_ref[...] += jnp.sum(jnp.where(valid, x, 0.0), axis=1, keepdims=True)

    tgt = tgt_ref[...]
    hit = cols == tgt
    pt_ref[...] += jnp.sum(jnp.where(hit, x, 0.0), axis=1, keepdims=True)

    @pl.when(i == nb - 1)
    def _finish():
        lse = m_ref[...] + jnp.log(s_ref[...])
        nonpad = tgt != PADDING_IDX
        loss_row = jnp.where(
            nonpad,
            C_ENT - SV * sum_ref[...] + SV * p0_ref[...]
            + (SV - CONFIDENCE) * pt_ref[...] + lse,
            0.0,
        )
        cnt = jnp.sum(nonpad.astype(jnp.float32))
        out_ref[...] = (jnp.sum(loss_row) / cnt).reshape(1, 1)


@jax.jit
def kernel(pred, target):
    n, vocab = pred.shape
    nb = pl.cdiv(vocab, BV)
    tgt2 = target.reshape(n, 1)
    out = pl.pallas_call(
        functools.partial(_loss_kernel, nb=nb, vocab=vocab),
        grid=(nb,),
        in_specs=[
            pl.BlockSpec((n, 1), lambda i: (0, 0)),
            pl.BlockSpec((n, BV), lambda i: (0, i)),
        ],
        out_specs=pl.BlockSpec((1, 1), lambda i: (0, 0)),
        out_shape=jax.ShapeDtypeStruct((1, 1), jnp.float32),
        scratch_shapes=[pltpu.VMEM((n, 1), jnp.float32) for _ in range(5)],
    )(tgt2, pred)
    return out[0, 0]
